# Initial kernel scaffold; baseline (speedup 1.0000x reference)
import jax
import jax.numpy as jnp
from jax.experimental import pallas as pl


def kernel(reward, probs, not_done):
    raise NotImplementedError("write your pallas kernel here")


# speedup vs baseline: 41.7195x; 41.7195x over previous
import functools

import jax
import jax.numpy as jnp
from jax import lax
from jax.experimental import pallas as pl
from jax.experimental.pallas import tpu as pltpu
from jax.experimental.pallas import tpu_sc as plsc

V_MIN = -10.0
V_MAX = 10.0
NUM_ATOMS = 51
DISCOUNT = 0.99
ATOM_DELTA = (V_MAX - V_MIN) / (NUM_ATOMS - 1)
INV_DELTA = 2.5

NC = 2
NS = 16
NW = NC * NS
LANES = 16


def _sc_body(bs, rows_w, rew_hbm, nd_hbm, probs_hbm, out_hbm,
             rew_v, nd_v, probs_v, out_v):
    wid = lax.axis_index("s") * NC + lax.axis_index("c")
    base = wid * rows_w
    words_w = rows_w * NUM_ATOMS

    pltpu.sync_copy(rew_hbm.at[pl.ds(base, rows_w)], rew_v)
    pltpu.sync_copy(nd_hbm.at[pl.ds(base, rows_w)], nd_v)
    pltpu.sync_copy(probs_hbm.at[pl.ds(base * NUM_ATOMS, words_w)], probs_v)

    iota = lax.iota(jnp.int32, LANES)
    iota51 = iota * NUM_ATOMS
    zeros16 = jnp.zeros((LANES,), jnp.float32)
    nblocks = rows_w // LANES

    def block(b, _):
        r16 = b * LANES
        rew = rew_v[pl.ds(r16, LANES)]
        g = nd_v[pl.ds(r16, LANES)] * DISCOUNT
        blk_base = b * (LANES * NUM_ATOMS)
        rows51 = iota51 + blk_base
        for k in range(NUM_ATOMS):
            out_v[pl.ds(blk_base + k * LANES, LANES)] = zeros16
        for j in range(NUM_ATOMS):
            a_j = V_MIN + ATOM_DELTA * j
            p = plsc.load_gather(probs_v, [rows51 + j])
            val = rew + g * a_j
            val = jnp.minimum(jnp.maximum(val, V_MIN), V_MAX)
            xf = (val - V_MIN) * INV_DELTA
            li = xf.astype(jnp.int32)
            frac = xf - li.astype(jnp.float32)
            uv = frac * p
            lv = p - uv
            ui = jnp.minimum(li + 1, NUM_ATOMS - 1)
            plsc.addupdate_scatter(out_v, [rows51 + li], lv)
            plsc.addupdate_scatter(out_v, [rows51 + ui], uv)
        return _

    lax.fori_loop(0, nblocks, block, None)
    pltpu.sync_copy(out_v, out_hbm.at[pl.ds(base * NUM_ATOMS, words_w)])


@jax.jit
def kernel(reward, probs, not_done):
    bs = probs.shape[0]
    rows_w = bs // NW
    words_w = rows_w * NUM_ATOMS
    mesh = plsc.VectorSubcoreMesh(
        core_axis_name="c", subcore_axis_name="s",
        num_cores=NC, num_subcores=NS)
    run = pl.kernel(
        functools.partial(_sc_body, bs, rows_w),
        out_type=jax.ShapeDtypeStruct((bs * NUM_ATOMS,), jnp.float32),
        mesh=mesh,
        compiler_params=pltpu.CompilerParams(needs_layout_passes=False),
        scratch_types=[
            pltpu.VMEM((rows_w,), jnp.float32),
            pltpu.VMEM((rows_w,), jnp.float32),
            pltpu.VMEM((words_w,), jnp.float32),
            pltpu.VMEM((words_w,), jnp.float32),
        ],
    )
    flat = run(reward.reshape(-1), not_done.reshape(-1), probs.reshape(-1))
    return flat.reshape(bs, NUM_ATOMS)
